# Initial kernel scaffold; baseline (speedup 1.0000x reference)
import jax
import jax.numpy as jnp
from jax.experimental import pallas as pl


def kernel(x, w1, b1, w2, b2):
    raise NotImplementedError("write your pallas kernel here")


# speedup vs baseline: 1.1335x; 1.1335x over previous
import functools

import jax
import jax.numpy as jnp
from jax.experimental import pallas as pl
from jax.experimental.pallas import tpu as pltpu


def _resblock_body(mask_ref, x_ref, w1_ref, b1_ref, w2_ref, b2_ref, o_ref,
                   *, W, nb):
    HW = x_ref.shape[2]

    m_xm = mask_ref[0:1, :]
    m_xp = mask_ref[1:2, :]
    m_yu = mask_ref[2:3, :]
    m_yd = mask_ref[3:4, :]
    b1 = b1_ref[...]
    b2 = b2_ref[...]

    def conv(inp, w_ref):
        cm = pltpu.roll(inp, shift=1, axis=1) * m_xm
        cp = pltpu.roll(inp, shift=HW - 1, axis=1) * m_xp
        xc = jnp.concatenate([cm, inp, cp], axis=0)
        up = pltpu.roll(xc, shift=W, axis=1) * m_yu
        dn = pltpu.roll(xc, shift=HW - W, axis=1) * m_yd
        acc = jnp.dot(w_ref[0], up, preferred_element_type=jnp.float32)
        acc = acc + jnp.dot(w_ref[1], xc, preferred_element_type=jnp.float32)
        acc = acc + jnp.dot(w_ref[2], dn, preferred_element_type=jnp.float32)
        return acc

    for i in range(nb):
        x0 = x_ref[i]
        hidden = jnp.maximum(conv(x0, w1_ref) + b1, 0.0)
        y = conv(hidden, w2_ref) + b2 + x0
        o_ref[i] = y.astype(o_ref.dtype)


def _edge_masks(H, W, dtype=jnp.float32):
    q = jnp.arange(H * W, dtype=jnp.int32)
    w = q % W
    rows = [
        (w >= 1).astype(dtype),
        (w <= W - 2).astype(dtype),
        (q >= W).astype(dtype),
        (q < H * W - W).astype(dtype),
    ]
    rows += [jnp.zeros((H * W,), dtype)] * 4
    return jnp.stack(rows, axis=0)


def _row_grouped(wt, C):
    return jnp.transpose(wt, (2, 0, 3, 1)).reshape(3, C, 3 * C)


def kernel(x, w1, b1, w2, b2):
    N, C, H, W = x.shape
    HW = H * W

    nb = 8
    while N % nb:
        nb //= 2
    grid = (N // nb,)

    x_r = x.reshape(N, C, HW)
    wk1 = _row_grouped(w1, C)
    wk2 = _row_grouped(w2, C)
    b1_k = b1.reshape(C, 1)
    b2_k = b2.reshape(C, 1)
    masks = _edge_masks(H, W)

    body = functools.partial(_resblock_body, W=W, nb=nb)
    out = pl.pallas_call(
        body,
        out_shape=jax.ShapeDtypeStruct((N, C, HW), x.dtype),
        grid=grid,
        in_specs=[
            pl.BlockSpec((8, HW), lambda g: (0, 0)),
            pl.BlockSpec((nb, C, HW), lambda g: (g, 0, 0)),
            pl.BlockSpec((3, C, 3 * C), lambda g: (0, 0, 0)),
            pl.BlockSpec((C, 1), lambda g: (0, 0)),
            pl.BlockSpec((3, C, 3 * C), lambda g: (0, 0, 0)),
            pl.BlockSpec((C, 1), lambda g: (0, 0)),
        ],
        out_specs=pl.BlockSpec((nb, C, HW), lambda g: (g, 0, 0)),
        compiler_params=pltpu.CompilerParams(
            dimension_semantics=("parallel",),
            vmem_limit_bytes=48 << 20,
        ),
    )(masks, x_r, wk1, b1_k, wk2, b2_k)

    return out.reshape(N, C, H, W)


# speedup vs baseline: 1.4159x; 1.2491x over previous
import functools

import jax
import jax.numpy as jnp
from jax.experimental import pallas as pl
from jax.experimental.pallas import tpu as pltpu


def _resblock_body(mask_ref, x_ref, w1_ref, b1_ref, w2_ref, b2_ref, o_ref,
                   *, W, nb):
    C = x_ref.shape[1]
    HW = x_ref.shape[2]

    m_xm = mask_ref[0:1, :]
    m_xp = mask_ref[1:2, :]
    m_yu = mask_ref[2:3, :]
    m_yd = mask_ref[3:4, :]
    b1 = b1_ref[...]
    b2 = b2_ref[...]

    def conv(inp, w_ref):
        cm = pltpu.roll(inp, shift=1, axis=1) * m_xm
        cp = pltpu.roll(inp, shift=HW - 1, axis=1) * m_xp
        xc = jnp.concatenate([cm, inp, cp], axis=0)
        p = jnp.dot(w_ref[...], xc, preferred_element_type=jnp.float32)
        acc = (p[C:2 * C]
               + pltpu.roll(p[:C], shift=W, axis=1) * m_yu
               + pltpu.roll(p[2 * C:], shift=HW - W, axis=1) * m_yd)
        return acc

    for i in range(nb):
        x0 = x_ref[i]
        hidden = jnp.maximum(conv(x0, w1_ref) + b1, 0.0)
        y = conv(hidden, w2_ref) + b2 + x0
        o_ref[i] = y.astype(o_ref.dtype)


def _edge_masks(H, W, dtype=jnp.float32):
    q = jnp.arange(H * W, dtype=jnp.int32)
    w = q % W
    rows = [
        (w >= 1).astype(dtype),
        (w <= W - 2).astype(dtype),
        (q >= W).astype(dtype),
        (q < H * W - W).astype(dtype),
    ]
    rows += [jnp.zeros((H * W,), dtype)] * 4
    return jnp.stack(rows, axis=0)


def _row_grouped(wt, C):
    return jnp.transpose(wt, (2, 0, 3, 1)).reshape(3 * C, 3 * C)


def kernel(x, w1, b1, w2, b2):
    N, C, H, W = x.shape
    HW = H * W

    nb = 8
    while N % nb:
        nb //= 2
    grid = (N // nb,)

    x_r = x.reshape(N, C, HW)
    wk1 = _row_grouped(w1, C)
    wk2 = _row_grouped(w2, C)
    b1_k = b1.reshape(C, 1)
    b2_k = b2.reshape(C, 1)
    masks = _edge_masks(H, W)

    body = functools.partial(_resblock_body, W=W, nb=nb)
    out = pl.pallas_call(
        body,
        out_shape=jax.ShapeDtypeStruct((N, C, HW), x.dtype),
        grid=grid,
        in_specs=[
            pl.BlockSpec((8, HW), lambda g: (0, 0)),
            pl.BlockSpec((nb, C, HW), lambda g: (g, 0, 0)),
            pl.BlockSpec((3 * C, 3 * C), lambda g: (0, 0)),
            pl.BlockSpec((C, 1), lambda g: (0, 0)),
            pl.BlockSpec((3 * C, 3 * C), lambda g: (0, 0)),
            pl.BlockSpec((C, 1), lambda g: (0, 0)),
        ],
        out_specs=pl.BlockSpec((nb, C, HW), lambda g: (g, 0, 0)),
        compiler_params=pltpu.CompilerParams(
            dimension_semantics=("parallel",),
            vmem_limit_bytes=48 << 20,
        ),
    )(masks, x_r, wk1, b1_k, wk2, b2_k)

    return out.reshape(N, C, H, W)


# speedup vs baseline: 1.8502x; 1.3068x over previous
import functools

import jax
import jax.numpy as jnp
from jax.experimental import pallas as pl
from jax.experimental.pallas import tpu as pltpu


def _resblock_body(mask_ref, x_ref, w1_ref, b1_ref, w2_ref, b2_ref, o_ref,
                   *, W, nb):
    C = x_ref.shape[1]
    HW = x_ref.shape[2]

    m_xm = mask_ref[0:1, :]
    m_xp = mask_ref[1:2, :]
    m_yu = mask_ref[2:3, :]
    m_yd = mask_ref[3:4, :]
    b1 = b1_ref[...]
    b2 = b2_ref[...]

    def shifted(val_bf16, shift, mask):
        v = pltpu.bitcast(val_bf16, jnp.int32)
        v = pltpu.roll(v, shift=shift, axis=1) & mask
        return pltpu.bitcast(v, jnp.bfloat16)

    def conv(inp_bf, w_ref):
        cm = shifted(inp_bf, 1, m_xm)
        cp = shifted(inp_bf, HW - 1, m_xp)
        xc = jnp.concatenate([cm, inp_bf, cp], axis=0)
        p = jnp.dot(w_ref[...], xc, preferred_element_type=jnp.float32)
        up = shifted(p[:C].astype(jnp.bfloat16), W, m_yu)
        dn = shifted(p[2 * C:].astype(jnp.bfloat16), HW - W, m_yd)
        return p[C:2 * C] + (up + dn).astype(jnp.float32)

    for i in range(nb):
        x0 = x_ref[i]
        hidden = jnp.maximum(conv(x0.astype(jnp.bfloat16), w1_ref) + b1, 0.0)
        y = conv(hidden.astype(jnp.bfloat16), w2_ref) + b2 + x0
        o_ref[i] = y.astype(o_ref.dtype)


def _edge_masks(H, W):
    q = jnp.arange(H * W, dtype=jnp.int32)
    w = q % W
    rows = [
        w >= 1,
        w <= W - 2,
        q >= W,
        q < H * W - W,
    ]
    rows = [jnp.where(r, jnp.int32(-1), jnp.int32(0)) for r in rows]
    rows += [jnp.zeros((H * W,), jnp.int32)] * 4
    return jnp.stack(rows, axis=0)


def _row_grouped(wt, C):
    return jnp.transpose(wt, (2, 0, 3, 1)).reshape(3 * C, 3 * C).astype(
        jnp.bfloat16)


def kernel(x, w1, b1, w2, b2):
    N, C, H, W = x.shape
    HW = H * W

    nb = 8
    while N % nb:
        nb //= 2
    grid = (N // nb,)

    x_r = x.reshape(N, C, HW)
    wk1 = _row_grouped(w1, C)
    wk2 = _row_grouped(w2, C)
    b1_k = b1.reshape(C, 1)
    b2_k = b2.reshape(C, 1)
    masks = _edge_masks(H, W)

    body = functools.partial(_resblock_body, W=W, nb=nb)
    out = pl.pallas_call(
        body,
        out_shape=jax.ShapeDtypeStruct((N, C, HW), x.dtype),
        grid=grid,
        in_specs=[
            pl.BlockSpec((8, HW), lambda g: (0, 0)),
            pl.BlockSpec((nb, C, HW), lambda g: (g, 0, 0)),
            pl.BlockSpec((3 * C, 3 * C), lambda g: (0, 0)),
            pl.BlockSpec((C, 1), lambda g: (0, 0)),
            pl.BlockSpec((3 * C, 3 * C), lambda g: (0, 0)),
            pl.BlockSpec((C, 1), lambda g: (0, 0)),
        ],
        out_specs=pl.BlockSpec((nb, C, HW), lambda g: (g, 0, 0)),
        compiler_params=pltpu.CompilerParams(
            dimension_semantics=("parallel",),
            vmem_limit_bytes=48 << 20,
        ),
    )(masks, x_r, wk1, b1_k, wk2, b2_k)

    return out.reshape(N, C, H, W)


# speedup vs baseline: 2.1088x; 1.1398x over previous
import functools

import jax
import jax.numpy as jnp
from jax.experimental import pallas as pl
from jax.experimental.pallas import tpu as pltpu


def _resblock_body(mask_ref, x_ref, w1_ref, b1_ref, w2_ref, b2_ref, o_ref,
                   *, W, nb, G):
    C = x_ref.shape[1]
    HW = x_ref.shape[2]
    GHW = G * HW

    m_xm = mask_ref[0:1, :]
    m_xp = mask_ref[1:2, :]
    m_yu = mask_ref[2:3, :]
    m_yd = mask_ref[3:4, :]
    b1 = b1_ref[...]
    b2 = b2_ref[...]

    def conv(inp_bf, w_ref):
        v = pltpu.bitcast(inp_bf, jnp.int32)
        cm = pltpu.bitcast(pltpu.roll(v, shift=1, axis=1) & m_xm,
                           jnp.bfloat16)
        cp = pltpu.bitcast(pltpu.roll(v, shift=GHW - 1, axis=1) & m_xp,
                           jnp.bfloat16)
        xc = jnp.concatenate([cm, inp_bf, cp], axis=0)
        p = jnp.dot(w_ref[...], xc, preferred_element_type=jnp.float32)
        p0 = pltpu.bitcast(p[:C].astype(jnp.bfloat16), jnp.int32)
        p2 = pltpu.bitcast(p[2 * C:].astype(jnp.bfloat16), jnp.int32)
        up = pltpu.bitcast(pltpu.roll(p0, shift=W, axis=1) & m_yu,
                           jnp.bfloat16)
        dn = pltpu.bitcast(pltpu.roll(p2, shift=GHW - W, axis=1) & m_yd,
                           jnp.bfloat16)
        return p[C:2 * C].astype(jnp.bfloat16) + (up + dn)

    for i in range(nb // G):
        x0 = jnp.concatenate([x_ref[G * i + j] for j in range(G)], axis=1)
        hidden = jnp.maximum(conv(x0.astype(jnp.bfloat16), w1_ref) + b1, 0)
        y = x0 + conv(hidden, w2_ref).astype(jnp.float32) + b2
        y = y.astype(o_ref.dtype)
        for j in range(G):
            o_ref[G * i + j] = y[:, j * HW:(j + 1) * HW]


def _edge_masks(H, W, G):
    q = jnp.arange(H * W, dtype=jnp.int32)
    w = q % W
    rows = [
        w >= 1,
        w <= W - 2,
        q >= W,
        q < H * W - W,
    ]
    rows = [jnp.where(r, jnp.int32(-1), jnp.int32(0)) for r in rows]
    rows += [jnp.zeros((H * W,), jnp.int32)] * 4
    return jnp.tile(jnp.stack(rows, axis=0), (1, G))


def _row_grouped(wt, C):
    return jnp.transpose(wt, (2, 0, 3, 1)).reshape(3 * C, 3 * C).astype(
        jnp.bfloat16)


def kernel(x, w1, b1, w2, b2):
    N, C, H, W = x.shape
    HW = H * W

    nb = 8
    while N % nb:
        nb //= 2
    G = 2 if nb % 2 == 0 else 1
    grid = (N // nb,)

    x_r = x.reshape(N, C, HW)
    wk1 = _row_grouped(w1, C)
    wk2 = _row_grouped(w2, C)
    b1_k = b1.reshape(C, 1).astype(jnp.bfloat16)
    b2_k = b2.reshape(C, 1)
    masks = _edge_masks(H, W, G)

    body = functools.partial(_resblock_body, W=W, nb=nb, G=G)
    out = pl.pallas_call(
        body,
        out_shape=jax.ShapeDtypeStruct((N, C, HW), x.dtype),
        grid=grid,
        in_specs=[
            pl.BlockSpec((8, G * HW), lambda g: (0, 0)),
            pl.BlockSpec((nb, C, HW), lambda g: (g, 0, 0)),
            pl.BlockSpec((3 * C, 3 * C), lambda g: (0, 0)),
            pl.BlockSpec((C, 1), lambda g: (0, 0)),
            pl.BlockSpec((3 * C, 3 * C), lambda g: (0, 0)),
            pl.BlockSpec((C, 1), lambda g: (0, 0)),
        ],
        out_specs=pl.BlockSpec((nb, C, HW), lambda g: (g, 0, 0)),
        compiler_params=pltpu.CompilerParams(
            dimension_semantics=("parallel",),
            vmem_limit_bytes=48 << 20,
        ),
    )(masks, x_r, wk1, b1_k, wk2, b2_k)

    return out.reshape(N, C, H, W)


# speedup vs baseline: 2.1886x; 1.0379x over previous
import functools

import jax
import jax.numpy as jnp
from jax.experimental import pallas as pl
from jax.experimental.pallas import tpu as pltpu


def _resblock_body(mask_ref, x_ref, w1_ref, b1_ref, w2_ref, b2_ref, o_ref,
                   *, W, nb, G):
    C = x_ref.shape[1]
    HW = x_ref.shape[2]
    GHW = G * HW

    m_xm = mask_ref[0:1, :]
    m_xp = mask_ref[1:2, :]
    m_yu = mask_ref[2:3, :]
    m_yd = mask_ref[3:4, :]
    b1 = b1_ref[...]
    b2 = b2_ref[...]

    def conv(inp_bf, w_ref):
        v = pltpu.bitcast(inp_bf, jnp.int32)
        cm = pltpu.bitcast(pltpu.roll(v, shift=1, axis=1) & m_xm,
                           jnp.bfloat16)
        cp = pltpu.bitcast(pltpu.roll(v, shift=GHW - 1, axis=1) & m_xp,
                           jnp.bfloat16)
        xc = jnp.concatenate([cm, inp_bf, cp], axis=0)
        p = jnp.dot(w_ref[...], xc, preferred_element_type=jnp.float32)
        p0 = pltpu.bitcast(p[:C].astype(jnp.bfloat16), jnp.int32)
        p2 = pltpu.bitcast(p[2 * C:].astype(jnp.bfloat16), jnp.int32)
        up = pltpu.bitcast(pltpu.roll(p0, shift=W, axis=1) & m_yu,
                           jnp.bfloat16)
        dn = pltpu.bitcast(pltpu.roll(p2, shift=GHW - W, axis=1) & m_yd,
                           jnp.bfloat16)
        return p[C:2 * C].astype(jnp.bfloat16) + (up + dn)

    for i in range(nb // G):
        x0 = jnp.concatenate([x_ref[G * i + j] for j in range(G)], axis=1)
        hidden = jnp.maximum(conv(x0.astype(jnp.bfloat16), w1_ref) + b1, 0)
        y = x0 + conv(hidden, w2_ref).astype(jnp.float32) + b2
        y = y.astype(o_ref.dtype)
        for j in range(G):
            o_ref[G * i + j] = y[:, j * HW:(j + 1) * HW]


def _edge_masks(H, W, G):
    q = jnp.arange(H * W, dtype=jnp.int32)
    w = q % W
    rows = [
        w >= 1,
        w <= W - 2,
        q >= W,
        q < H * W - W,
    ]
    rows = [jnp.where(r, jnp.int32(-1), jnp.int32(0)) for r in rows]
    rows += [jnp.zeros((H * W,), jnp.int32)] * 4
    return jnp.tile(jnp.stack(rows, axis=0), (1, G))


def _row_grouped(wt, C):
    return jnp.transpose(wt, (2, 0, 3, 1)).reshape(3 * C, 3 * C).astype(
        jnp.bfloat16)


def kernel(x, w1, b1, w2, b2):
    N, C, H, W = x.shape
    HW = H * W

    nb = 8
    while N % nb:
        nb //= 2
    G = 4 if nb % 4 == 0 else (2 if nb % 2 == 0 else 1)
    grid = (N // nb,)

    x_r = x.reshape(N, C, HW)
    wk1 = _row_grouped(w1, C)
    wk2 = _row_grouped(w2, C)
    b1_k = b1.reshape(C, 1).astype(jnp.bfloat16)
    b2_k = b2.reshape(C, 1)
    masks = _edge_masks(H, W, G)

    body = functools.partial(_resblock_body, W=W, nb=nb, G=G)
    out = pl.pallas_call(
        body,
        out_shape=jax.ShapeDtypeStruct((N, C, HW), x.dtype),
        grid=grid,
        in_specs=[
            pl.BlockSpec((8, G * HW), lambda g: (0, 0)),
            pl.BlockSpec((nb, C, HW), lambda g: (g, 0, 0)),
            pl.BlockSpec((3 * C, 3 * C), lambda g: (0, 0)),
            pl.BlockSpec((C, 1), lambda g: (0, 0)),
            pl.BlockSpec((3 * C, 3 * C), lambda g: (0, 0)),
            pl.BlockSpec((C, 1), lambda g: (0, 0)),
        ],
        out_specs=pl.BlockSpec((nb, C, HW), lambda g: (g, 0, 0)),
        compiler_params=pltpu.CompilerParams(
            dimension_semantics=("parallel",),
            vmem_limit_bytes=48 << 20,
        ),
    )(masks, x_r, wk1, b1_k, wk2, b2_k)

    return out.reshape(N, C, H, W)
